# baseline (speedup 1.0000x reference)
import functools

import jax
import jax.numpy as jnp
from jax import lax
from jax.experimental import pallas as pl
from jax.experimental.pallas import tpu as pltpu
from jax.experimental.pallas import tpu_sc as plsc

VOCAB = 1000000
DIM = 64
NC, NS = 2, 16
NW = NC * NS
NROW, NCOL = 4096, 200
RPW = NROW // NW
RCH = 2
NCHUNK = RPW // RCH
SPLITS = ((0, 128), (128, 72))


def _embed_lookup(x, table):
    mesh = plsc.VectorSubcoreMesh(core_axis_name="c", subcore_axis_name="s")

    @functools.partial(
        pl.kernel,
        out_type=jax.ShapeDtypeStruct((NROW, NCOL, DIM), jnp.float32),
        mesh=mesh,
        scratch_types=[
            pltpu.VMEM((2, RCH, NCOL), jnp.int32),
            pltpu.VMEM((2, RCH, NCOL, DIM), jnp.float32),
            pltpu.SemaphoreType.DMA,
            pltpu.SemaphoreType.DMA,
            pltpu.SemaphoreType.DMA,
        ],
        compiler_params=pltpu.CompilerParams(use_tc_tiling_on_sc=False),
    )
    def body(x_hbm, table_hbm, out_hbm, idx_v, rows_v, gsem0, gsem1, osem):
        wid = lax.axis_index("s") * NC + lax.axis_index("c")
        row0 = wid * RPW
        gsems = (gsem0, gsem1)

        def stage_idx(i, b):
            pltpu.sync_copy(x_hbm.at[pl.ds(row0 + i * RCH, RCH)], idx_v.at[b])

        def gather_copies(i, b):
            copies = []
            for j in range(RCH):
                for off, n in SPLITS:
                    copies.append(pltpu.make_async_copy(
                        table_hbm.at[idx_v.at[b].at[j].at[pl.ds(off, n)]],
                        rows_v.at[b].at[j].at[pl.ds(off, n)],
                        gsems[b],
                    ))
            return copies

        def fire_gathers(i, b):
            for c in gather_copies(i, b):
                c.start()

        def drain_gathers(i, b):
            for c in gather_copies(i, b):
                c.wait()

        def out_copy(i, b):
            return pltpu.make_async_copy(
                rows_v.at[b],
                out_hbm.at[pl.ds(row0 + i * RCH, RCH)],
                osem,
            )

        stage_idx(0, 0)
        fire_gathers(0, 0)

        def outer(t, carry):
            for b in range(2):
                i = t * 2 + b

                @pl.when(i > 0)
                def _():
                    out_copy(i - 1, 1 - b).wait()

                @pl.when(i + 1 < NCHUNK)
                def _():
                    stage_idx(i + 1, 1 - b)
                    fire_gathers(i + 1, 1 - b)

                drain_gathers(i, b)
                out_copy(i, b).start()
            return carry

        lax.fori_loop(0, NCHUNK // 2, outer, 0)
        out_copy(NCHUNK - 1, 1).wait()

    return body(x, table)


def kernel(x, table):
    return _embed_lookup(x.astype(jnp.int32), table)


# speedup vs baseline: 1.0153x; 1.0153x over previous
import functools

import jax
import jax.numpy as jnp
from jax import lax
from jax.experimental import pallas as pl
from jax.experimental.pallas import tpu as pltpu
from jax.experimental.pallas import tpu_sc as plsc

VOCAB = 1000000
DIM = 64
NC, NS = 2, 16
NW = NC * NS
B = 4096 * 200
BPW = B // NW
SUB = 128
K = 4
CHUNK = SUB * K
NCHUNK = BPW // CHUNK
IDXROWS = BPW // SUB


def _embed_lookup(x2d, table):
    mesh = plsc.VectorSubcoreMesh(core_axis_name="c", subcore_axis_name="s")

    @functools.partial(
        pl.kernel,
        out_type=jax.ShapeDtypeStruct((B, DIM), jnp.float32),
        mesh=mesh,
        scratch_types=[
            pltpu.VMEM((IDXROWS, SUB), jnp.int32),
            pltpu.VMEM((2, CHUNK, DIM), jnp.float32),
            pltpu.SemaphoreType.DMA,
            pltpu.SemaphoreType.DMA,
            pltpu.SemaphoreType.DMA,
        ],
        compiler_params=pltpu.CompilerParams(use_tc_tiling_on_sc=False),
    )
    def body(x_hbm, table_hbm, out_hbm, idx_v, rows_v, gsem0, gsem1, osem):
        wid = lax.axis_index("s") * NC + lax.axis_index("c")
        row0 = wid * IDXROWS
        gsems = (gsem0, gsem1)
        table_rows = table_hbm

        pltpu.sync_copy(x_hbm.at[pl.ds(row0, IDXROWS)], idx_v)

        def fire_gathers(i, b):
            for j in range(K):
                pltpu.async_copy(
                    table_rows.at[idx_v.at[i * K + j]],
                    rows_v.at[b].at[pl.ds(j * SUB, SUB)],
                    gsems[b],
                )

        def drain_gathers(i, b):
            for j in range(K):
                pltpu.make_async_copy(
                    table_rows.at[idx_v.at[i * K + j]],
                    rows_v.at[b].at[pl.ds(j * SUB, SUB)],
                    gsems[b],
                ).wait()

        def fire_out(i, b):
            pltpu.async_copy(
                rows_v.at[b],
                out_hbm.at[pl.ds(row0 * SUB + i * CHUNK, CHUNK)],
                osem,
            )

        def drain_out(i, b):
            pltpu.make_async_copy(
                rows_v.at[b],
                out_hbm.at[pl.ds(row0 * SUB + i * CHUNK, CHUNK)],
                osem,
            ).wait()

        fire_gathers(0, 0)

        def outer(t, carry):
            for b in range(2):
                i = t * 2 + b

                @pl.when(i > 0)
                def _():
                    drain_out(i - 1, 1 - b)

                @pl.when(i + 1 < NCHUNK)
                def _():
                    fire_gathers(i + 1, 1 - b)

                drain_gathers(i, b)
                fire_out(i, b)
            return carry

        lax.fori_loop(0, NCHUNK // 2, outer, 0)
        drain_out(NCHUNK - 1, 1)

    return body(x2d, table)


def kernel(x, table):
    x2d = x.reshape(B // SUB, SUB).astype(jnp.int32)
    out = _embed_lookup(x2d, table)
    return out.reshape(4096, 200, DIM)
